# baseline (speedup 1.0000x reference)
import jax
import jax.numpy as jnp
from jax import lax
from jax.experimental import pallas as pl
from jax.experimental.pallas import tpu as pltpu
from jax.experimental.pallas import tpu_sc as plsc

N_USERS = 100000
N_ITEMS = 100000
DIM = 128
B = 16384
DECAY = 1e-4

NC = 2
NS = 16
NW = NC * NS
BPW = B // NW
CHUNK = 64
NCHUNK = BPW // CHUNK
UCHUNK = 128
NUCHUNK = BPW // UCHUNK
LANES = 16
VPR = DIM // LANES
PROWS = BPW * LANES // DIM


def _sc_body(uemb, iemb, uidx, pidx, nidx,
             d_out, sq_out,
             idx_u, idx_pn, urows, pnrows, dbuf, sqbuf, sem_u, sem_pn):
    wid = lax.axis_index("s") * NC + lax.axis_index("c")
    base = wid * BPW

    cps_i = [pltpu.async_copy(uidx.at[pl.ds(base, BPW)], idx_u, sem_u)]
    for c in range(NCHUNK):
        cps_i.append(pltpu.async_copy(
            pidx.at[pl.ds(base + c * CHUNK, CHUNK)],
            idx_pn.at[pl.ds(c * 2 * CHUNK, CHUNK)], sem_pn))
        cps_i.append(pltpu.async_copy(
            nidx.at[pl.ds(base + c * CHUNK, CHUNK)],
            idx_pn.at[pl.ds(c * 2 * CHUNK + CHUNK, CHUNK)], sem_pn))
    for cp in cps_i:
        cp.wait()

    zero = jnp.zeros((LANES,), jnp.float32)

    def gather_u(uc):
        return pltpu.async_copy(
            uemb.at[idx_u.at[pl.ds(uc * UCHUNK, UCHUNK)]],
            urows.at[uc % 2], sem_u)

    def gather_pn(c):
        return pltpu.async_copy(
            iemb.at[idx_pn.at[pl.ds(c * 2 * CHUNK, 2 * CHUNK)]],
            pnrows.at[c % 3], sem_pn)

    pend_u = [gather_u(0)] + [None] * (NUCHUNK - 1)
    pend_pn = [gather_pn(0), gather_pn(1)] + [None] * (NCHUNK - 2)
    sq_acc = tuple(zero for _ in range(3 * VPR))

    for c in range(NCHUNK):
        uc, uoff = c // 2, (c % 2) * CHUNK
        if c % 2 == 0:
            pend_u[uc].wait()
            if uc + 1 < NUCHUNK:
                pend_u[uc + 1] = gather_u(uc + 1)
        pend_pn[c].wait()
        if c + 2 < NCHUNK:
            pend_pn[c + 2] = gather_pn(c + 2)
        ubuf, buf = uc % 2, c % 3

        def row_body(r, carry):
            sq = list(carry)
            uv = [urows[ubuf, uoff + r, pl.ds(j * LANES, LANES)]
                  for j in range(VPR)]
            pv = [pnrows[buf, r, pl.ds(j * LANES, LANES)] for j in range(VPR)]
            nv = [pnrows[buf, CHUNK + r, pl.ds(j * LANES, LANES)]
                  for j in range(VPR)]
            td = [uv[j] * (pv[j] - nv[j]) for j in range(VPR)]
            w = VPR
            while w > 1:
                w //= 2
                td = [td[j] + td[j + w] for j in range(w)]
            for j in range(VPR):
                sq[j] = sq[j] + uv[j] * uv[j]
                sq[VPR + j] = sq[VPR + j] + pv[j] * pv[j]
                sq[2 * VPR + j] = sq[2 * VPR + j] + nv[j] * nv[j]
            dbuf[c * (CHUNK // VPR) + r // VPR,
                 pl.ds((r % VPR) * LANES, LANES)] = td[0]
            return tuple(sq)

        sq_acc = lax.fori_loop(0, CHUNK, row_body, sq_acc)

    pltpu.sync_copy(dbuf, d_out.at[pl.ds(wid * PROWS, PROWS)])
    su = sum(sq_acc[0:VPR], zero)
    sp = sum(sq_acc[VPR:2 * VPR], zero)
    sn = sum(sq_acc[2 * VPR:3 * VPR], zero)
    sqbuf[pl.ds(0, LANES)] = su
    sqbuf[pl.ds(LANES, LANES)] = sp
    sqbuf[pl.ds(2 * LANES, LANES)] = sn
    pltpu.sync_copy(sqbuf, sq_out.at[wid])


def _tc_body(d_ref, sq_ref, out_ref):
    dm = d_ref[...]
    kk = lax.broadcasted_iota(jnp.int32, (DIM, DIM // LANES), 0) // LANES
    jj = lax.broadcasted_iota(jnp.int32, (DIM, DIM // LANES), 1)
    sel = (kk == jj).astype(jnp.float32)
    d = jnp.dot(dm, sel, preferred_element_type=jnp.float32)
    bpr = jnp.sum(-jnp.log(1e-5 + jax.nn.sigmoid(d))) / B
    sq = sq_ref[...]
    s_u = jnp.sum(sq[:, 0:16])
    s_p = jnp.sum(sq[:, 16:32])
    s_n = jnp.sum(sq[:, 32:48])
    emb = (jnp.sqrt(s_u) + jnp.sqrt(s_p) + jnp.sqrt(s_n)) / B * DECAY
    out_ref[...] = jnp.reshape(bpr + emb / B, (1, 1))


@jax.jit
def kernel(users, pos_items, neg_items, user_emb, item_emb):
    uidx = users.astype(jnp.int32)
    pidx = pos_items.astype(jnp.int32)
    nidx = neg_items[:, 0].astype(jnp.int32)

    sc = pl.kernel(
        _sc_body,
        mesh=plsc.VectorSubcoreMesh(core_axis_name="c", subcore_axis_name="s"),
        out_type=(
            jax.ShapeDtypeStruct((B * LANES // DIM, DIM), jnp.float32),
            jax.ShapeDtypeStruct((NW, 3 * LANES), jnp.float32),
        ),
        scratch_types=[
            pltpu.VMEM((BPW,), jnp.int32),
            pltpu.VMEM((2 * BPW,), jnp.int32),
            pltpu.VMEM((2, UCHUNK, DIM), jnp.float32),
            pltpu.VMEM((3, 2 * CHUNK, DIM), jnp.float32),
            pltpu.VMEM((PROWS, DIM), jnp.float32),
            pltpu.VMEM((3 * LANES,), jnp.float32),
            pltpu.SemaphoreType.DMA,
            pltpu.SemaphoreType.DMA,
        ],
    )
    d_part, sq = sc(user_emb, item_emb, uidx, pidx, nidx)

    res = pl.pallas_call(
        _tc_body,
        out_shape=jax.ShapeDtypeStruct((1, 1), jnp.float32),
    )(d_part, sq)

    s = res[0, 0]
    return (s, s, s)


# speedup vs baseline: 1.0141x; 1.0141x over previous
import functools

import jax
import jax.numpy as jnp
from jax import lax
from jax.experimental import pallas as pl
from jax.experimental.pallas import tpu as pltpu
from jax.experimental.pallas import tpu_sc as plsc

N_USERS = 100000
N_ITEMS = 100000
DIM = 128
B = 16384
DECAY = 1e-4

NC = 2
NS = 16
NW = NC * NS
BPW = B // NW
CHUNK = 64
NCHUNK = BPW // CHUNK
NBUF = 3
LANES = 16
VPR = DIM // LANES
PROWS = BPW * LANES // DIM


def _sc_body(uemb, iemb, uidx, pidx, nidx,
             d_out, sq_out,
             idx_u, idx_p, idx_n, urows, prows, nrows, dbuf, sqbuf, sem, osem):
    wid = lax.axis_index("s") * NC + lax.axis_index("c")
    base = wid * BPW

    ci = pltpu.async_copy(uidx.at[pl.ds(base, BPW)], idx_u, sem)
    cj = pltpu.async_copy(pidx.at[pl.ds(base, BPW)], idx_p, sem)
    ck = pltpu.async_copy(nidx.at[pl.ds(base, BPW)], idx_n, sem)
    ci.wait()
    cj.wait()
    ck.wait()

    zero = jnp.zeros((LANES,), jnp.float32)

    def gather(c, buf):
        s = pl.ds(c * CHUNK, CHUNK)
        return (pltpu.async_copy(uemb.at[idx_u.at[s]], urows.at[buf], sem),
                pltpu.async_copy(iemb.at[idx_p.at[s]], prows.at[buf], sem),
                pltpu.async_copy(iemb.at[idx_n.at[s]], nrows.at[buf], sem))

    pend = [gather(0, 0), gather(1, 1)] + [None] * (NCHUNK - 2)
    sq_acc = tuple(zero for _ in range(3 * VPR))

    for c in range(NCHUNK):
        for cp in pend[c]:
            cp.wait()
        if c + 2 < NCHUNK:
            pend[c + 2] = gather(c + 2, (c + 2) % NBUF)
        buf = c % NBUF

        def row_body(r, carry):
            sq = list(carry)
            uv = [urows[buf, r, pl.ds(j * LANES, LANES)] for j in range(VPR)]
            pv = [prows[buf, r, pl.ds(j * LANES, LANES)] for j in range(VPR)]
            nv = [nrows[buf, r, pl.ds(j * LANES, LANES)] for j in range(VPR)]
            td = [uv[j] * (pv[j] - nv[j]) for j in range(VPR)]
            w = VPR
            while w > 1:
                w //= 2
                td = [td[j] + td[j + w] for j in range(w)]
            for j in range(VPR):
                sq[j] = sq[j] + uv[j] * uv[j]
                sq[VPR + j] = sq[VPR + j] + pv[j] * pv[j]
                sq[2 * VPR + j] = sq[2 * VPR + j] + nv[j] * nv[j]
            dbuf[c * (CHUNK // VPR) + r // VPR,
                 pl.ds((r % VPR) * LANES, LANES)] = td[0]
            return tuple(sq)

        sq_acc = lax.fori_loop(0, CHUNK, row_body, sq_acc)
        if c == NCHUNK // 2 - 1:
            half = PROWS // 2
            out_cp = pltpu.async_copy(
                dbuf.at[pl.ds(0, half)],
                d_out.at[pl.ds(wid * PROWS, half)], osem)

    half = PROWS // 2
    pltpu.sync_copy(dbuf.at[pl.ds(half, half)],
                    d_out.at[pl.ds(wid * PROWS + half, half)])
    out_cp.wait()
    su = sum(sq_acc[0:VPR], zero)
    sp = sum(sq_acc[VPR:2 * VPR], zero)
    sn = sum(sq_acc[2 * VPR:3 * VPR], zero)
    sqbuf[pl.ds(0, LANES)] = su
    sqbuf[pl.ds(LANES, LANES)] = sp
    sqbuf[pl.ds(2 * LANES, LANES)] = sn
    pltpu.sync_copy(sqbuf, sq_out.at[wid])


def _tc_body(d_ref, sq_ref, out_ref):
    dm = d_ref[...]
    kk = lax.broadcasted_iota(jnp.int32, (DIM, DIM // LANES), 0) // LANES
    jj = lax.broadcasted_iota(jnp.int32, (DIM, DIM // LANES), 1)
    sel = (kk == jj).astype(jnp.float32)
    d = jnp.dot(dm, sel, preferred_element_type=jnp.float32)
    bpr = jnp.sum(-jnp.log(1e-5 + jax.nn.sigmoid(d))) / B
    sq = sq_ref[...]
    s_u = jnp.sum(sq[:, 0:16])
    s_p = jnp.sum(sq[:, 16:32])
    s_n = jnp.sum(sq[:, 32:48])
    emb = (jnp.sqrt(s_u) + jnp.sqrt(s_p) + jnp.sqrt(s_n)) / B * DECAY
    out_ref[...] = jnp.reshape(bpr + emb / B, (1, 1))


@jax.jit
def kernel(users, pos_items, neg_items, user_emb, item_emb):
    uidx = users.astype(jnp.int32)
    pidx = pos_items.astype(jnp.int32)
    nidx = neg_items[:, 0].astype(jnp.int32)

    sc = pl.kernel(
        _sc_body,
        mesh=plsc.VectorSubcoreMesh(core_axis_name="c", subcore_axis_name="s"),
        out_type=(
            jax.ShapeDtypeStruct((B * LANES // DIM, DIM), jnp.float32),
            jax.ShapeDtypeStruct((NW, 3 * LANES), jnp.float32),
        ),
        scratch_types=[
            pltpu.VMEM((BPW,), jnp.int32),
            pltpu.VMEM((BPW,), jnp.int32),
            pltpu.VMEM((BPW,), jnp.int32),
            pltpu.VMEM((NBUF, CHUNK, DIM), jnp.float32),
            pltpu.VMEM((NBUF, CHUNK, DIM), jnp.float32),
            pltpu.VMEM((NBUF, CHUNK, DIM), jnp.float32),
            pltpu.VMEM((PROWS, DIM), jnp.float32),
            pltpu.VMEM((3 * LANES,), jnp.float32),
            pltpu.SemaphoreType.DMA,
            pltpu.SemaphoreType.DMA,
        ],
    )
    d_part, sq = sc(user_emb, item_emb, uidx, pidx, nidx)

    res = pl.pallas_call(
        _tc_body,
        out_shape=jax.ShapeDtypeStruct((1, 1), jnp.float32),
    )(d_part, sq)

    s = res[0, 0]
    return (s, s, s)


# speedup vs baseline: 1.0165x; 1.0024x over previous
import jax
import jax.numpy as jnp
from jax import lax
from jax.experimental import pallas as pl
from jax.experimental.pallas import tpu as pltpu
from jax.experimental.pallas import tpu_sc as plsc

N_USERS = 100000
N_ITEMS = 100000
DIM = 128
B = 16384
DECAY = 1e-4

NC = 2
NS = 16
NW = NC * NS
BPW = B // NW
CHUNK = 64
NCHUNK = BPW // CHUNK
NBUF = 3
LANES = 16
VPR = DIM // LANES
PROWS = BPW * LANES // DIM


def _sc_body(uemb, iemb, uidx, pidx, nidx,
             d_out, sq_out,
             idx_u, idx_p, idx_n, urows, prows, nrows, dbuf, sqbuf, sem, osem):
    wid = lax.axis_index("s") * NC + lax.axis_index("c")
    base = wid * BPW

    ci = pltpu.async_copy(uidx.at[pl.ds(base, BPW)], idx_u, sem)
    cj = pltpu.async_copy(pidx.at[pl.ds(base, BPW)], idx_p, sem)
    ck = pltpu.async_copy(nidx.at[pl.ds(base, BPW)], idx_n, sem)
    ci.wait()
    cj.wait()
    ck.wait()

    zero = jnp.zeros((LANES,), jnp.float32)

    def gather(c, buf):
        s = pl.ds(c * CHUNK, CHUNK)
        return (pltpu.async_copy(uemb.at[idx_u.at[s]], urows.at[buf], sem),
                pltpu.async_copy(iemb.at[idx_p.at[s]], prows.at[buf], sem),
                pltpu.async_copy(iemb.at[idx_n.at[s]], nrows.at[buf], sem))

    pend = [gather(0, 0), gather(1, 1)] + [None] * (NCHUNK - 2)
    sq_acc = tuple(zero for _ in range(3 * VPR))

    for c in range(NCHUNK):
        for cp in pend[c]:
            cp.wait()
        if c + 2 < NCHUNK:
            pend[c + 2] = gather(c + 2, (c + 2) % NBUF)
        buf = c % NBUF

        def row_body(r, carry):
            sq = list(carry)
            uv = [urows[buf, r, pl.ds(j * LANES, LANES)] for j in range(VPR)]
            pv = [prows[buf, r, pl.ds(j * LANES, LANES)] for j in range(VPR)]
            nv = [nrows[buf, r, pl.ds(j * LANES, LANES)] for j in range(VPR)]
            td = [uv[j] * (pv[j] - nv[j]) for j in range(VPR)]
            w = VPR
            while w > 1:
                w //= 2
                td = [td[j] + td[j + w] for j in range(w)]
            for j in range(VPR):
                sq[j] = sq[j] + uv[j] * uv[j]
                sq[VPR + j] = sq[VPR + j] + pv[j] * pv[j]
                sq[2 * VPR + j] = sq[2 * VPR + j] + nv[j] * nv[j]
            dbuf[c * (CHUNK // VPR) + r // VPR,
                 pl.ds((r % VPR) * LANES, LANES)] = td[0]
            return tuple(sq)

        sq_acc = lax.fori_loop(0, CHUNK, row_body, sq_acc)
        if c == NCHUNK // 2 - 1:
            half = PROWS // 2
            out_cp = pltpu.async_copy(
                dbuf.at[pl.ds(0, half)],
                d_out.at[pl.ds(wid * PROWS, half)], osem)

    half = PROWS // 2
    pltpu.sync_copy(dbuf.at[pl.ds(half, half)],
                    d_out.at[pl.ds(wid * PROWS + half, half)])
    out_cp.wait()
    su = sum(sq_acc[0:VPR], zero)
    sp = sum(sq_acc[VPR:2 * VPR], zero)
    sn = sum(sq_acc[2 * VPR:3 * VPR], zero)
    sqbuf[pl.ds(0, LANES)] = su
    sqbuf[pl.ds(LANES, LANES)] = sp
    sqbuf[pl.ds(2 * LANES, LANES)] = sn
    pltpu.sync_copy(sqbuf, sq_out.at[wid])


def _tc_body(d_ref, sq_ref, out_ref):
    dm = d_ref[...]
    kk = lax.broadcasted_iota(jnp.int32, (DIM, DIM // LANES), 0) // LANES
    jj = lax.broadcasted_iota(jnp.int32, (DIM, DIM // LANES), 1)
    sel = (kk == jj).astype(jnp.float32)
    d = jnp.dot(dm, sel, preferred_element_type=jnp.float32)
    bpr = jnp.sum(-jnp.log(1e-5 + jax.nn.sigmoid(d))) / B
    sq = sq_ref[...]
    s_u = jnp.sum(sq[:, 0:16])
    s_p = jnp.sum(sq[:, 16:32])
    s_n = jnp.sum(sq[:, 32:48])
    emb = (jnp.sqrt(s_u) + jnp.sqrt(s_p) + jnp.sqrt(s_n)) / B * DECAY
    out_ref[...] = jnp.reshape(bpr + emb / B, (1, 1))


@jax.jit
def kernel(users, pos_items, neg_items, user_emb, item_emb):
    uidx = users.astype(jnp.int32)
    pidx = pos_items.astype(jnp.int32)
    nidx = neg_items[:, 0].astype(jnp.int32)

    sc = pl.kernel(
        _sc_body,
        mesh=plsc.VectorSubcoreMesh(core_axis_name="c", subcore_axis_name="s"),
        out_type=(
            jax.ShapeDtypeStruct((B * LANES // DIM, DIM), jnp.float32),
            jax.ShapeDtypeStruct((NW, 3 * LANES), jnp.float32),
        ),
        scratch_types=[
            pltpu.VMEM((BPW,), jnp.int32),
            pltpu.VMEM((BPW,), jnp.int32),
            pltpu.VMEM((BPW,), jnp.int32),
            pltpu.VMEM((NBUF, CHUNK, DIM), jnp.float32),
            pltpu.VMEM((NBUF, CHUNK, DIM), jnp.float32),
            pltpu.VMEM((NBUF, CHUNK, DIM), jnp.float32),
            pltpu.VMEM((PROWS, DIM), jnp.float32),
            pltpu.VMEM((3 * LANES,), jnp.float32),
            pltpu.SemaphoreType.DMA,
            pltpu.SemaphoreType.DMA,
        ],
    )
    d_part, sq = sc(user_emb, item_emb, uidx, pidx, nidx)

    res = pl.pallas_call(
        _tc_body,
        out_shape=jax.ShapeDtypeStruct((1, 1), jnp.float32),
    )(d_part, sq)

    s = res[0, 0]
    return (s, s, s)
